# baseline (speedup 1.0000x reference)
import functools

import jax
import jax.numpy as jnp
from jax import lax
from jax.experimental import pallas as pl
from jax.experimental.pallas import tpu as pltpu
from jax.experimental.pallas import tpu_sc as plsc

N = 10000
E = 160000
NODE_IN = 128
H = 32
H2 = H * H

TN = 2000
TE = 640

NC = 2
NS = 16
NW = NC * NS
CH = 128
NCHUNK = E // CH
CPW = -(-NCHUNK // NW)
ZR = N // NS

_PREC = lax.Precision.HIGHEST


def _dot(a, b):
    return jnp.dot(a, b, preferred_element_type=jnp.float32, precision=_PREC)


def _h_body(x_ref, w_ref, b_ref, o_ref):
    o_ref[...] = jnp.maximum(_dot(x_ref[...], w_ref[...]) + b_ref[...], 0.0)


def _compute_h(x, W_in, b_in):
    return pl.pallas_call(
        _h_body,
        grid=(N // TN,),
        in_specs=[
            pl.BlockSpec((TN, NODE_IN), lambda i: (i, 0)),
            pl.BlockSpec((NODE_IN, H), lambda i: (0, 0)),
            pl.BlockSpec((1, H), lambda i: (0, 0)),
        ],
        out_specs=pl.BlockSpec((TN, H), lambda i: (i, 0)),
        out_shape=jax.ShapeDtypeStruct((N, H), jnp.float32),
    )(x, W_in, b_in.reshape(1, H))


_SC_MESH = plsc.VectorSubcoreMesh(core_axis_name="c", subcore_axis_name="s")


def _gather_body(h_hbm, src_hbm, out_hbm, idx_v, rows_v, sem):
    wid = lax.axis_index("s") * NC + lax.axis_index("c")

    def body(g, carry):
        chunk = wid + g * NW

        @pl.when(chunk < NCHUNK)
        def _():
            pltpu.sync_copy(src_hbm.at[chunk], idx_v)
            pltpu.async_copy(h_hbm.at[idx_v], rows_v, sem).wait()
            pltpu.sync_copy(rows_v, out_hbm.at[pl.ds(chunk * CH, CH)])

        return carry

    lax.fori_loop(0, CPW, body, 0)


_gather_call = pl.kernel(
    _gather_body,
    out_type=jax.ShapeDtypeStruct((E, H), jnp.float32),
    mesh=_SC_MESH,
    scratch_types=[
        pltpu.VMEM((CH,), jnp.int32),
        pltpu.VMEM((CH, H), jnp.float32),
        pltpu.SemaphoreType.DMA,
    ],
    compiler_params=pltpu.CompilerParams(use_tc_tiling_on_sc=False),
)


def _msg_body(ea_ref, hs_ref, w1t_ref, b1t_ref, r_ref, p_ref, b2m_ref, o_ref):
    tbig = jnp.maximum(_dot(ea_ref[...], w1t_ref[...]) + b1t_ref[...], 0.0)
    hrep = _dot(hs_ref[...], r_ref[...])
    o_ref[...] = _dot(tbig * hrep, p_ref[...]) + _dot(hs_ref[...], b2m_ref[...])


def _compute_msg(edge_attr, hsrc, W1t, b1t, Rmat, Pmat, B2m):
    return pl.pallas_call(
        _msg_body,
        grid=(E // TE,),
        in_specs=[
            pl.BlockSpec((TE, 16), lambda i: (i, 0)),
            pl.BlockSpec((TE, H), lambda i: (i, 0)),
            pl.BlockSpec((16, H2), lambda i: (0, 0)),
            pl.BlockSpec((1, H2), lambda i: (0, 0)),
            pl.BlockSpec((H, H2), lambda i: (0, 0)),
            pl.BlockSpec((H2, H), lambda i: (0, 0)),
            pl.BlockSpec((H, H), lambda i: (0, 0)),
        ],
        out_specs=pl.BlockSpec((TE, H), lambda i: (i, 0)),
        out_shape=jax.ShapeDtypeStruct((E, H), jnp.float32),
    )(edge_attr, hsrc, W1t, b1t, Rmat, Pmat, B2m)


def _scatter_body(dst_hbm, msg_hbm, zrows_hbm, zn_hbm, agg_hbm, cnt_hbm,
                  idx_v, rows_v, ones_v, stage_v, cstage_v, acc_sh, cnt_sh, sem):
    cid = lax.axis_index("c")
    sid = lax.axis_index("s")
    wid = sid * NC + cid

    for i in range(CH // 16):
        ones_v[pl.ds(i * 16, 16)] = jnp.ones((16,), jnp.float32)

    pltpu.sync_copy(zrows_hbm, stage_v)
    pltpu.sync_copy(stage_v, acc_sh.at[pl.ds(sid * ZR, ZR)])

    @pl.when(sid == 0)
    def _():
        pltpu.sync_copy(zn_hbm, cstage_v)
        pltpu.sync_copy(cstage_v, cnt_sh)

    plsc.subcore_barrier()

    def body(g, carry):
        chunk = wid + g * NW

        @pl.when(chunk < NCHUNK)
        def _():
            pltpu.sync_copy(dst_hbm.at[chunk], idx_v)
            pltpu.sync_copy(msg_hbm.at[pl.ds(chunk * CH, CH)], rows_v)
            pltpu.sync_copy(rows_v, acc_sh.at[idx_v], add=True)
            pltpu.sync_copy(ones_v, cnt_sh.at[idx_v], add=True)

        return carry

    lax.fori_loop(0, CPW, body, 0)
    plsc.subcore_barrier()

    pltpu.sync_copy(acc_sh.at[pl.ds(sid * ZR, ZR)], stage_v)
    pltpu.sync_copy(stage_v, agg_hbm.at[cid, pl.ds(sid * ZR, ZR)])

    @pl.when(sid == 0)
    def _():
        pltpu.sync_copy(cnt_sh, cstage_v)
        pltpu.sync_copy(cstage_v, cnt_hbm.at[cid])


_scatter_call = pl.kernel(
    _scatter_body,
    out_type=(
        jax.ShapeDtypeStruct((NC, N, H), jnp.float32),
        jax.ShapeDtypeStruct((NC, N), jnp.float32),
    ),
    mesh=_SC_MESH,
    scratch_types=[
        pltpu.VMEM((CH,), jnp.int32),
        pltpu.VMEM((CH, H), jnp.float32),
        pltpu.VMEM((CH,), jnp.float32),
        pltpu.VMEM((ZR, H), jnp.float32),
        pltpu.VMEM((N,), jnp.float32),
        pltpu.VMEM_SHARED((N, H), jnp.float32),
        pltpu.VMEM_SHARED((N,), jnp.float32),
        pltpu.SemaphoreType.DMA,
    ],
    compiler_params=pltpu.CompilerParams(use_tc_tiling_on_sc=False),
)


def _final_body(h_ref, p0_ref, p1_ref, c0_ref, c1_ref, wr_ref, cb_ref,
                wir_ref, wiz_ref, win_ref, whr_ref, whz_ref, whn_ref,
                br_ref, bz_ref, bin_ref, bhn_ref, o_ref):
    h = h_ref[...]
    cnt = jnp.maximum(c0_ref[...] + c1_ref[...], 1.0)
    agg = (p0_ref[...] + p1_ref[...]) / cnt
    conv = agg + _dot(h, wr_ref[...]) + cb_ref[...]
    m = jnp.maximum(conv, 0.0)
    r = jax.nn.sigmoid(_dot(m, wir_ref[...]) + _dot(h, whr_ref[...]) + br_ref[...])
    z = jax.nn.sigmoid(_dot(m, wiz_ref[...]) + _dot(h, whz_ref[...]) + bz_ref[...])
    n = jnp.tanh(_dot(m, win_ref[...]) + bin_ref[...]
                 + r * (_dot(h, whn_ref[...]) + bhn_ref[...]))
    o_ref[...] = (1.0 - z) * n + z * h


def _compute_final(h, p0, p1, c0, c1, weights):
    node_spec = pl.BlockSpec((TN, H), lambda i: (i, 0))
    cnt_spec = pl.BlockSpec((TN, 1), lambda i: (i, 0))
    w_spec = pl.BlockSpec((H, H), lambda i: (0, 0))
    b_spec = pl.BlockSpec((1, H), lambda i: (0, 0))
    return pl.pallas_call(
        _final_body,
        grid=(N // TN,),
        in_specs=[node_spec, node_spec, node_spec, cnt_spec, cnt_spec,
                  w_spec, b_spec,
                  w_spec, w_spec, w_spec, w_spec, w_spec, w_spec,
                  b_spec, b_spec, b_spec, b_spec],
        out_specs=node_spec,
        out_shape=jax.ShapeDtypeStruct((N, H), jnp.float32),
    )(h, p0, p1, c0, c1, *weights)


def kernel(x, edge_index, edge_attr, W_in, b_in, W1, b1, W2, b2,
           W_root, conv_bias, W_ih, W_hh, b_ih, b_hh):
    src2d = edge_index[0].reshape(NCHUNK, CH)
    dst2d = edge_index[1].reshape(NCHUNK, CH)

    W1t = jnp.tile(W1, (1, H))
    b1t = jnp.tile(b1, H).reshape(1, H2)
    Rmat = jnp.repeat(jnp.eye(H, dtype=jnp.float32), H, axis=1)
    Pmat = W2.reshape(H, H, H).transpose(1, 0, 2).reshape(H2, H)
    B2m = b2.reshape(H, H)

    Wir, Wiz, Win = (W_ih[i * H:(i + 1) * H, :].T for i in range(3))
    Whr, Whz, Whn = (W_hh[i * H:(i + 1) * H, :].T for i in range(3))
    br = (b_ih[0:H] + b_hh[0:H]).reshape(1, H)
    bz = (b_ih[H:2 * H] + b_hh[H:2 * H]).reshape(1, H)
    bin_ = b_ih[2 * H:3 * H].reshape(1, H)
    bhn = b_hh[2 * H:3 * H].reshape(1, H)

    h = _compute_h(x, W_in, b_in)
    hsrc = _gather_call(h, src2d)
    msg = _compute_msg(edge_attr, hsrc, W1t, b1t, Rmat, Pmat, B2m)
    zrows = jnp.zeros((ZR, H), jnp.float32)
    zn = jnp.zeros((N,), jnp.float32)
    aggp, cntp = _scatter_call(dst2d, msg, zrows, zn)

    p0 = aggp[0]
    p1 = aggp[1]
    c0 = cntp[0].reshape(N, 1)
    c1 = cntp[1].reshape(N, 1)
    weights = (W_root, conv_bias.reshape(1, H),
               Wir, Wiz, Win, Whr, Whz, Whn, br, bz, bin_, bhn)
    return _compute_final(h, p0, p1, c0, c1, weights)


# speedup vs baseline: 3.9277x; 3.9277x over previous
import functools

import jax
import jax.numpy as jnp
from jax import lax
from jax.experimental import pallas as pl
from jax.experimental.pallas import tpu as pltpu
from jax.experimental.pallas import tpu_sc as plsc

N = 10000
E = 160000
NODE_IN = 128
H = 32
H2 = H * H

TN = 2000
TE = 640

NC = 2
NS = 16
NW = NC * NS
CH = 128
NCHUNK = E // CH
CPW = -(-NCHUNK // NW)
ZR = N // NS

_PREC = lax.Precision.HIGHEST


def _dot(a, b):
    return jnp.dot(a, b, preferred_element_type=jnp.float32, precision=_PREC)


def _h_body(x_ref, w_ref, b_ref, o_ref):
    o_ref[...] = jnp.maximum(_dot(x_ref[...], w_ref[...]) + b_ref[...], 0.0)


def _compute_h(x, W_in, b_in):
    return pl.pallas_call(
        _h_body,
        grid=(N // TN,),
        in_specs=[
            pl.BlockSpec((TN, NODE_IN), lambda i: (i, 0)),
            pl.BlockSpec((NODE_IN, H), lambda i: (0, 0)),
            pl.BlockSpec((1, H), lambda i: (0, 0)),
        ],
        out_specs=pl.BlockSpec((TN, H), lambda i: (i, 0)),
        out_shape=jax.ShapeDtypeStruct((N, H), jnp.float32),
    )(x, W_in, b_in.reshape(1, H))


_SC_MESH = plsc.VectorSubcoreMesh(core_axis_name="c", subcore_axis_name="s")


def _gather_body(h_hbm, src_hbm, out_hbm, idx_v, rows_v, sem):
    wid = lax.axis_index("s") * NC + lax.axis_index("c")

    def body(g, carry):
        chunk = wid + g * NW

        @pl.when(chunk < NCHUNK)
        def _():
            pltpu.sync_copy(src_hbm.at[chunk], idx_v)
            pltpu.async_copy(h_hbm.at[idx_v], rows_v, sem).wait()
            pltpu.sync_copy(rows_v, out_hbm.at[pl.ds(chunk * CH, CH)])

        return carry

    lax.fori_loop(0, CPW, body, 0)


_gather_call = pl.kernel(
    _gather_body,
    out_type=jax.ShapeDtypeStruct((E, H), jnp.float32),
    mesh=_SC_MESH,
    scratch_types=[
        pltpu.VMEM((CH,), jnp.int32),
        pltpu.VMEM((CH, H), jnp.float32),
        pltpu.SemaphoreType.DMA,
    ],
    compiler_params=pltpu.CompilerParams(use_tc_tiling_on_sc=False),
)


def _msg_body(ea_ref, hs_ref, w1r_ref, b1r_ref, p2_ref, b2m_ref, o_ref):
    tbig = jnp.maximum(
        jnp.dot(ea_ref[...], w1r_ref[...], preferred_element_type=jnp.float32)
        + b1r_ref[...], 0.0)
    hs = hs_ref[...]
    u = tbig * jnp.tile(hs, (1, H))
    o_ref[...] = (jnp.dot(u, p2_ref[...], preferred_element_type=jnp.float32)
                  + _dot(hs, b2m_ref[...]))


def _compute_msg(edge_attr, hsrc, W1r, b1r, P2, B2m):
    return pl.pallas_call(
        _msg_body,
        grid=(E // TE,),
        in_specs=[
            pl.BlockSpec((TE, 16), lambda i: (i, 0)),
            pl.BlockSpec((TE, H), lambda i: (i, 0)),
            pl.BlockSpec((16, H2), lambda i: (0, 0)),
            pl.BlockSpec((1, H2), lambda i: (0, 0)),
            pl.BlockSpec((H2, H), lambda i: (0, 0)),
            pl.BlockSpec((H, H), lambda i: (0, 0)),
        ],
        out_specs=pl.BlockSpec((TE, H), lambda i: (i, 0)),
        out_shape=jax.ShapeDtypeStruct((E, H), jnp.float32),
    )(edge_attr, hsrc, W1r, b1r, P2, B2m)


def _scatter_body(dst_hbm, msg_hbm, zrows_hbm, zn_hbm, agg_hbm, cnt_hbm,
                  idx_v, rows_v, ones_v, stage_v, cstage_v, acc_sh, cnt_sh, sem):
    cid = lax.axis_index("c")
    sid = lax.axis_index("s")
    wid = sid * NC + cid

    for i in range(CH // 16):
        ones_v[pl.ds(i * 16, 16)] = jnp.ones((16,), jnp.float32)

    pltpu.sync_copy(zrows_hbm, stage_v)
    pltpu.sync_copy(stage_v, acc_sh.at[pl.ds(sid * ZR, ZR)])

    @pl.when(sid == 0)
    def _():
        pltpu.sync_copy(zn_hbm, cstage_v)
        pltpu.sync_copy(cstage_v, cnt_sh)

    plsc.subcore_barrier()

    def body(g, carry):
        chunk = wid + g * NW

        @pl.when(chunk < NCHUNK)
        def _():
            pltpu.sync_copy(dst_hbm.at[chunk], idx_v)
            pltpu.sync_copy(msg_hbm.at[pl.ds(chunk * CH, CH)], rows_v)
            pltpu.sync_copy(rows_v, acc_sh.at[idx_v], add=True)
            pltpu.sync_copy(ones_v, cnt_sh.at[idx_v], add=True)

        return carry

    lax.fori_loop(0, CPW, body, 0)
    plsc.subcore_barrier()

    pltpu.sync_copy(acc_sh.at[pl.ds(sid * ZR, ZR)], stage_v)
    pltpu.sync_copy(stage_v, agg_hbm.at[cid, pl.ds(sid * ZR, ZR)])

    @pl.when(sid == 0)
    def _():
        pltpu.sync_copy(cnt_sh, cstage_v)
        pltpu.sync_copy(cstage_v, cnt_hbm.at[cid])


_scatter_call = pl.kernel(
    _scatter_body,
    out_type=(
        jax.ShapeDtypeStruct((NC, N, H), jnp.float32),
        jax.ShapeDtypeStruct((NC, N), jnp.float32),
    ),
    mesh=_SC_MESH,
    scratch_types=[
        pltpu.VMEM((CH,), jnp.int32),
        pltpu.VMEM((CH, H), jnp.float32),
        pltpu.VMEM((CH,), jnp.float32),
        pltpu.VMEM((ZR, H), jnp.float32),
        pltpu.VMEM((N,), jnp.float32),
        pltpu.VMEM_SHARED((N, H), jnp.float32),
        pltpu.VMEM_SHARED((N,), jnp.float32),
        pltpu.SemaphoreType.DMA,
    ],
    compiler_params=pltpu.CompilerParams(use_tc_tiling_on_sc=False),
)


def _final_body(h_ref, p0_ref, p1_ref, c0_ref, c1_ref, wr_ref, cb_ref,
                wir_ref, wiz_ref, win_ref, whr_ref, whz_ref, whn_ref,
                br_ref, bz_ref, bin_ref, bhn_ref, o_ref):
    h = h_ref[...]
    cnt = jnp.maximum(c0_ref[...] + c1_ref[...], 1.0)
    agg = (p0_ref[...] + p1_ref[...]) / cnt
    conv = agg + _dot(h, wr_ref[...]) + cb_ref[...]
    m = jnp.maximum(conv, 0.0)
    r = jax.nn.sigmoid(_dot(m, wir_ref[...]) + _dot(h, whr_ref[...]) + br_ref[...])
    z = jax.nn.sigmoid(_dot(m, wiz_ref[...]) + _dot(h, whz_ref[...]) + bz_ref[...])
    n = jnp.tanh(_dot(m, win_ref[...]) + bin_ref[...]
                 + r * (_dot(h, whn_ref[...]) + bhn_ref[...]))
    o_ref[...] = (1.0 - z) * n + z * h


def _compute_final(h, p0, p1, c0, c1, weights):
    node_spec = pl.BlockSpec((TN, H), lambda i: (i, 0))
    cnt_spec = pl.BlockSpec((TN, 1), lambda i: (i, 0))
    w_spec = pl.BlockSpec((H, H), lambda i: (0, 0))
    b_spec = pl.BlockSpec((1, H), lambda i: (0, 0))
    return pl.pallas_call(
        _final_body,
        grid=(N // TN,),
        in_specs=[node_spec, node_spec, node_spec, cnt_spec, cnt_spec,
                  w_spec, b_spec,
                  w_spec, w_spec, w_spec, w_spec, w_spec, w_spec,
                  b_spec, b_spec, b_spec, b_spec],
        out_specs=node_spec,
        out_shape=jax.ShapeDtypeStruct((N, H), jnp.float32),
    )(h, p0, p1, c0, c1, *weights)


def kernel(x, edge_index, edge_attr, W_in, b_in, W1, b1, W2, b2,
           W_root, conv_bias, W_ih, W_hh, b_ih, b_hh):
    src2d = edge_index[0].reshape(NCHUNK, CH)
    dst2d = edge_index[1].reshape(NCHUNK, CH)

    W1r = jnp.repeat(W1, H, axis=1)
    b1r = jnp.repeat(b1, H).reshape(1, H2)
    P2 = W2.reshape(H2, H)
    B2m = b2.reshape(H, H)

    Wir, Wiz, Win = (W_ih[i * H:(i + 1) * H, :].T for i in range(3))
    Whr, Whz, Whn = (W_hh[i * H:(i + 1) * H, :].T for i in range(3))
    br = (b_ih[0:H] + b_hh[0:H]).reshape(1, H)
    bz = (b_ih[H:2 * H] + b_hh[H:2 * H]).reshape(1, H)
    bin_ = b_ih[2 * H:3 * H].reshape(1, H)
    bhn = b_hh[2 * H:3 * H].reshape(1, H)

    h = _compute_h(x, W_in, b_in)
    hsrc = _gather_call(h, src2d)
    msg = _compute_msg(edge_attr, hsrc, W1r, b1r, P2, B2m)
    zrows = jnp.zeros((ZR, H), jnp.float32)
    zn = jnp.zeros((N,), jnp.float32)
    aggp, cntp = _scatter_call(dst2d, msg, zrows, zn)

    p0 = aggp[0]
    p1 = aggp[1]
    c0 = cntp[0].reshape(N, 1)
    c1 = cntp[1].reshape(N, 1)
    weights = (W_root, conv_bias.reshape(1, H),
               Wir, Wiz, Win, Whr, Whz, Whn, br, bz, bin_, bhn)
    return _compute_final(h, p0, p1, c0, c1, weights)


# speedup vs baseline: 4.7692x; 1.2143x over previous
import functools

import jax
import jax.numpy as jnp
from jax import lax
from jax.experimental import pallas as pl
from jax.experimental.pallas import tpu as pltpu
from jax.experimental.pallas import tpu_sc as plsc

N = 10000
E = 160000
NODE_IN = 128
H = 32
H2 = H * H

TN = 2000
TE = 1280

NC = 2
NS = 16
NW = NC * NS
CH = 128
NCHUNK = E // CH
CPW = -(-NCHUNK // NW)
ZR = N // NS

_PREC = lax.Precision.DEFAULT


def _dot(a, b):
    return jnp.dot(a, b, preferred_element_type=jnp.float32, precision=_PREC)


def _h_body(x_ref, w_ref, b_ref, o_ref):
    o_ref[...] = jnp.maximum(_dot(x_ref[...], w_ref[...]) + b_ref[...], 0.0)


def _compute_h(x, W_in, b_in):
    return pl.pallas_call(
        _h_body,
        grid=(N // TN,),
        in_specs=[
            pl.BlockSpec((TN, NODE_IN), lambda i: (i, 0)),
            pl.BlockSpec((NODE_IN, H), lambda i: (0, 0)),
            pl.BlockSpec((1, H), lambda i: (0, 0)),
        ],
        out_specs=pl.BlockSpec((TN, H), lambda i: (i, 0)),
        out_shape=jax.ShapeDtypeStruct((N, H), jnp.float32),
    )(x, W_in, b_in.reshape(1, H))


_SC_MESH = plsc.VectorSubcoreMesh(core_axis_name="c", subcore_axis_name="s")


def _gather_body(h_hbm, src_hbm, out_hbm, idx_v, rows_v, sem):
    wid = lax.axis_index("s") * NC + lax.axis_index("c")

    def body(g, carry):
        chunk = wid + g * NW

        @pl.when(chunk < NCHUNK)
        def _():
            pltpu.sync_copy(src_hbm.at[chunk], idx_v)
            pltpu.async_copy(h_hbm.at[idx_v], rows_v, sem).wait()
            pltpu.sync_copy(rows_v, out_hbm.at[pl.ds(chunk * CH, CH)])

        return carry

    lax.fori_loop(0, CPW, body, 0)


_gather_call = pl.kernel(
    _gather_body,
    out_type=jax.ShapeDtypeStruct((E, H), jnp.float32),
    mesh=_SC_MESH,
    scratch_types=[
        pltpu.VMEM((CH,), jnp.int32),
        pltpu.VMEM((CH, H), jnp.float32),
        pltpu.SemaphoreType.DMA,
    ],
    compiler_params=pltpu.CompilerParams(use_tc_tiling_on_sc=False),
)


def _msg_body(ea_ref, hs_ref, w1r_ref, b1r_ref, p2_ref, b2m_ref, o_ref):
    tbig = jnp.maximum(
        jnp.dot(ea_ref[...], w1r_ref[...], preferred_element_type=jnp.float32)
        + b1r_ref[...], 0.0)
    hs = hs_ref[...]
    u = (tbig * jnp.tile(hs, (1, H))).astype(jnp.bfloat16)
    o_ref[...] = (jnp.dot(u, p2_ref[...], preferred_element_type=jnp.float32)
                  + _dot(hs, b2m_ref[...]))


def _compute_msg(edge_attr, hsrc, W1r, b1r, P2, B2m):
    return pl.pallas_call(
        _msg_body,
        grid=(E // TE,),
        in_specs=[
            pl.BlockSpec((TE, 16), lambda i: (i, 0)),
            pl.BlockSpec((TE, H), lambda i: (i, 0)),
            pl.BlockSpec((16, H2), lambda i: (0, 0)),
            pl.BlockSpec((1, H2), lambda i: (0, 0)),
            pl.BlockSpec((H2, H), lambda i: (0, 0)),
            pl.BlockSpec((H, H), lambda i: (0, 0)),
        ],
        out_specs=pl.BlockSpec((TE, H), lambda i: (i, 0)),
        out_shape=jax.ShapeDtypeStruct((E, H), jnp.float32),
    )(edge_attr, hsrc, W1r, b1r, P2, B2m)


def _scatter_body(dst_hbm, msg_hbm, zrows_hbm, zn_hbm, agg_hbm, cnt_hbm,
                  idx_v, rows_v, ones_v, stage_v, cstage_v, acc_sh, cnt_sh, sem):
    cid = lax.axis_index("c")
    sid = lax.axis_index("s")
    wid = sid * NC + cid

    for i in range(CH // 16):
        ones_v[pl.ds(i * 16, 16)] = jnp.ones((16,), jnp.float32)

    pltpu.sync_copy(zrows_hbm, stage_v)
    pltpu.sync_copy(stage_v, acc_sh.at[pl.ds(sid * ZR, ZR)])

    @pl.when(sid == 0)
    def _():
        pltpu.sync_copy(zn_hbm, cstage_v)
        pltpu.sync_copy(cstage_v, cnt_sh)

    plsc.subcore_barrier()

    def body(g, carry):
        chunk = wid + g * NW

        @pl.when(chunk < NCHUNK)
        def _():
            pltpu.sync_copy(dst_hbm.at[chunk], idx_v)
            pltpu.sync_copy(msg_hbm.at[pl.ds(chunk * CH, CH)], rows_v)
            pltpu.sync_copy(rows_v, acc_sh.at[idx_v], add=True)
            pltpu.sync_copy(ones_v, cnt_sh.at[idx_v], add=True)

        return carry

    lax.fori_loop(0, CPW, body, 0)
    plsc.subcore_barrier()

    pltpu.sync_copy(acc_sh.at[pl.ds(sid * ZR, ZR)], stage_v)
    pltpu.sync_copy(stage_v, agg_hbm.at[cid, pl.ds(sid * ZR, ZR)])

    @pl.when(sid == 0)
    def _():
        pltpu.sync_copy(cnt_sh, cstage_v)
        pltpu.sync_copy(cstage_v, cnt_hbm.at[cid])


_scatter_call = pl.kernel(
    _scatter_body,
    out_type=(
        jax.ShapeDtypeStruct((NC, N, H), jnp.float32),
        jax.ShapeDtypeStruct((NC, N), jnp.float32),
    ),
    mesh=_SC_MESH,
    scratch_types=[
        pltpu.VMEM((CH,), jnp.int32),
        pltpu.VMEM((CH, H), jnp.float32),
        pltpu.VMEM((CH,), jnp.float32),
        pltpu.VMEM((ZR, H), jnp.float32),
        pltpu.VMEM((N,), jnp.float32),
        pltpu.VMEM_SHARED((N, H), jnp.float32),
        pltpu.VMEM_SHARED((N,), jnp.float32),
        pltpu.SemaphoreType.DMA,
    ],
    compiler_params=pltpu.CompilerParams(use_tc_tiling_on_sc=False),
)


def _final_body(h_ref, p0_ref, p1_ref, c0_ref, c1_ref, wr_ref, cb_ref,
                wir_ref, wiz_ref, win_ref, whr_ref, whz_ref, whn_ref,
                br_ref, bz_ref, bin_ref, bhn_ref, o_ref):
    h = h_ref[...]
    cnt = jnp.maximum(c0_ref[...] + c1_ref[...], 1.0)
    agg = (p0_ref[...] + p1_ref[...]) / cnt
    conv = agg + _dot(h, wr_ref[...]) + cb_ref[...]
    m = jnp.maximum(conv, 0.0)
    r = jax.nn.sigmoid(_dot(m, wir_ref[...]) + _dot(h, whr_ref[...]) + br_ref[...])
    z = jax.nn.sigmoid(_dot(m, wiz_ref[...]) + _dot(h, whz_ref[...]) + bz_ref[...])
    n = jnp.tanh(_dot(m, win_ref[...]) + bin_ref[...]
                 + r * (_dot(h, whn_ref[...]) + bhn_ref[...]))
    o_ref[...] = (1.0 - z) * n + z * h


def _compute_final(h, p0, p1, c0, c1, weights):
    node_spec = pl.BlockSpec((TN, H), lambda i: (i, 0))
    cnt_spec = pl.BlockSpec((TN, 1), lambda i: (i, 0))
    w_spec = pl.BlockSpec((H, H), lambda i: (0, 0))
    b_spec = pl.BlockSpec((1, H), lambda i: (0, 0))
    return pl.pallas_call(
        _final_body,
        grid=(N // TN,),
        in_specs=[node_spec, node_spec, node_spec, cnt_spec, cnt_spec,
                  w_spec, b_spec,
                  w_spec, w_spec, w_spec, w_spec, w_spec, w_spec,
                  b_spec, b_spec, b_spec, b_spec],
        out_specs=node_spec,
        out_shape=jax.ShapeDtypeStruct((N, H), jnp.float32),
    )(h, p0, p1, c0, c1, *weights)


def kernel(x, edge_index, edge_attr, W_in, b_in, W1, b1, W2, b2,
           W_root, conv_bias, W_ih, W_hh, b_ih, b_hh):
    src2d = edge_index[0].reshape(NCHUNK, CH)
    dst2d = edge_index[1].reshape(NCHUNK, CH)

    W1r = jnp.repeat(W1, H, axis=1).astype(jnp.bfloat16)
    ea_bf = edge_attr.astype(jnp.bfloat16)
    b1r = jnp.repeat(b1, H).reshape(1, H2)
    P2 = W2.reshape(H2, H).astype(jnp.bfloat16)
    B2m = b2.reshape(H, H)

    Wir, Wiz, Win = (W_ih[i * H:(i + 1) * H, :].T for i in range(3))
    Whr, Whz, Whn = (W_hh[i * H:(i + 1) * H, :].T for i in range(3))
    br = (b_ih[0:H] + b_hh[0:H]).reshape(1, H)
    bz = (b_ih[H:2 * H] + b_hh[H:2 * H]).reshape(1, H)
    bin_ = b_ih[2 * H:3 * H].reshape(1, H)
    bhn = b_hh[2 * H:3 * H].reshape(1, H)

    h = _compute_h(x, W_in, b_in)
    hsrc = _gather_call(h, src2d)
    msg = _compute_msg(ea_bf, hsrc, W1r, b1r, P2, B2m)
    zrows = jnp.zeros((ZR, H), jnp.float32)
    zn = jnp.zeros((N,), jnp.float32)
    aggp, cntp = _scatter_call(dst2d, msg, zrows, zn)

    p0 = aggp[0]
    p1 = aggp[1]
    c0 = cntp[0].reshape(N, 1)
    c1 = cntp[1].reshape(N, 1)
    weights = (W_root, conv_bias.reshape(1, H),
               Wir, Wiz, Win, Whr, Whz, Whn, br, bz, bin_, bhn)
    return _compute_final(h, p0, p1, c0, c1, weights)


# speedup vs baseline: 5.0676x; 1.0626x over previous
import jax
import jax.numpy as jnp
from jax import lax
from jax.experimental import pallas as pl
from jax.experimental.pallas import tpu as pltpu
from jax.experimental.pallas import tpu_sc as plsc

N = 10000
E = 160000
NODE_IN = 128
H = 32
H2 = H * H
W = 128

TN = 2000
TE = 1280

NC = 2
NS = 16
NW = NC * NS
CH = 128
NCHUNK = E // CH
CPW = -(-NCHUNK // NW)
NP = 10240
ZR = NP // NS

_PREC = lax.Precision.DEFAULT


def _dot(a, b):
    return jnp.dot(a, b, preferred_element_type=jnp.float32, precision=_PREC)


def _h_body(x_ref, w_ref, b_ref, o_ref):
    h = jnp.maximum(_dot(x_ref[...], w_ref[...]) + b_ref[...], 0.0)
    o_ref[...] = jnp.concatenate(
        [h, jnp.zeros((TN, W - H), jnp.float32)], axis=1)


def _compute_h(x, W_in, b_in):
    return pl.pallas_call(
        _h_body,
        grid=(N // TN,),
        in_specs=[
            pl.BlockSpec((TN, NODE_IN), lambda i: (i, 0)),
            pl.BlockSpec((NODE_IN, H), lambda i: (0, 0)),
            pl.BlockSpec((1, H), lambda i: (0, 0)),
        ],
        out_specs=pl.BlockSpec((TN, W), lambda i: (i, 0)),
        out_shape=jax.ShapeDtypeStruct((N, W), jnp.float32),
    )(x, W_in, b_in.reshape(1, H))


_SC_MESH = plsc.VectorSubcoreMesh(core_axis_name="c", subcore_axis_name="s")


def _gather_body(h_hbm, src_hbm, out_hbm, idx_v, rows_v, sem):
    wid = lax.axis_index("s") * NC + lax.axis_index("c")

    def body(g, carry):
        chunk = wid + g * NW

        @pl.when(chunk < NCHUNK)
        def _():
            pltpu.sync_copy(src_hbm.at[chunk], idx_v)
            pltpu.async_copy(h_hbm.at[idx_v], rows_v, sem).wait()
            pltpu.sync_copy(rows_v, out_hbm.at[pl.ds(chunk * CH, CH)])

        return carry

    lax.fori_loop(0, CPW, body, 0)


_gather_call = pl.kernel(
    _gather_body,
    out_type=jax.ShapeDtypeStruct((E, W), jnp.float32),
    mesh=_SC_MESH,
    scratch_types=[
        pltpu.VMEM((CH,), jnp.int32),
        pltpu.VMEM((CH, W), jnp.float32),
        pltpu.SemaphoreType.DMA,
    ],
)


def _msg_body(ea_ref, hs_ref, w1r_ref, b1r_ref, p2_ref, b2m_ref, o_ref):
    tbig = jnp.maximum(
        jnp.dot(ea_ref[...], w1r_ref[...], preferred_element_type=jnp.float32)
        + b1r_ref[...], 0.0)
    hs = hs_ref[:, :H]
    u = (tbig * jnp.tile(hs, (1, H))).astype(jnp.bfloat16)
    msg = (jnp.dot(u, p2_ref[...], preferred_element_type=jnp.float32)
           + _dot(hs, b2m_ref[...]))
    o_ref[...] = jnp.concatenate(
        [msg,
         jnp.ones((TE, 1), jnp.float32),
         jnp.zeros((TE, W - H - 1), jnp.float32)], axis=1)


def _compute_msg(edge_attr, hsrc, W1r, b1r, P2, B2m):
    return pl.pallas_call(
        _msg_body,
        grid=(E // TE,),
        in_specs=[
            pl.BlockSpec((TE, 16), lambda i: (i, 0)),
            pl.BlockSpec((TE, W), lambda i: (i, 0)),
            pl.BlockSpec((16, H2), lambda i: (0, 0)),
            pl.BlockSpec((1, H2), lambda i: (0, 0)),
            pl.BlockSpec((H2, H), lambda i: (0, 0)),
            pl.BlockSpec((H, H), lambda i: (0, 0)),
        ],
        out_specs=pl.BlockSpec((TE, W), lambda i: (i, 0)),
        out_shape=jax.ShapeDtypeStruct((E, W), jnp.float32),
    )(edge_attr, hsrc, W1r, b1r, P2, B2m)


HALF = NP // 2
HR = HALF + CH
ZR2 = HR // NS
OR2 = HALF // NS
CPW2 = -(-NCHUNK // NS)


def _clamp_idx(idx_ref, lo):
    for i in range(CH // 16):
        v = idx_ref[pl.ds(i * 16, 16)]
        rel = v - lo
        ok = (rel >= 0) & (rel < HALF)
        idx_ref[pl.ds(i * 16, 16)] = jnp.where(ok, rel, HALF)


def _scatter_body(dst_hbm, msg_hbm, zrows_hbm, agg_hbm,
                  idx0_v, idx1_v, rows0_v, rows1_v, acc_sh,
                  semi0, semr0, semi1, semr1):
    cid = lax.axis_index("c")
    sid = lax.axis_index("s")
    lo = cid * HALF

    pltpu.sync_copy(zrows_hbm, acc_sh.at[pl.ds(sid * ZR2, ZR2)])
    plsc.subcore_barrier()

    def chunk_of(j):
        return sid + j * NS

    def start(j, idx_v, rows_v, semi, semr):
        @pl.when(chunk_of(j) < NCHUNK)
        def _():
            c = chunk_of(j)
            pltpu.async_copy(dst_hbm.at[c], idx_v, semi)
            pltpu.async_copy(msg_hbm.at[pl.ds(c * CH, CH)], rows_v, semr)

    def drain_scatter(j, idx_v, rows_v, semi, semr):
        @pl.when(chunk_of(j) < NCHUNK)
        def _():
            pltpu.make_async_copy(dst_hbm.at[chunk_of(j)], idx_v, semi).wait()
            pltpu.make_async_copy(
                msg_hbm.at[pl.ds(chunk_of(j) * CH, CH)], rows_v, semr).wait()
            _clamp_idx(idx_v, lo)
            pltpu.sync_copy(rows_v, acc_sh.at[idx_v], add=True)

    start(0, idx0_v, rows0_v, semi0, semr0)

    def body(k, carry):
        start(2 * k + 1, idx1_v, rows1_v, semi1, semr1)
        drain_scatter(2 * k, idx0_v, rows0_v, semi0, semr0)
        start(2 * k + 2, idx0_v, rows0_v, semi0, semr0)
        drain_scatter(2 * k + 1, idx1_v, rows1_v, semi1, semr1)
        return carry

    lax.fori_loop(0, (CPW2 + 1) // 2, body, 0)
    plsc.subcore_barrier()

    pltpu.sync_copy(acc_sh.at[pl.ds(sid * OR2, OR2)],
                    agg_hbm.at[pl.ds(cid * HALF + sid * OR2, OR2)])


_scatter_call = pl.kernel(
    _scatter_body,
    out_type=jax.ShapeDtypeStruct((NP, W), jnp.float32),
    mesh=_SC_MESH,
    scratch_types=[
        pltpu.VMEM((CH,), jnp.int32),
        pltpu.VMEM((CH,), jnp.int32),
        pltpu.VMEM((CH, W), jnp.float32),
        pltpu.VMEM((CH, W), jnp.float32),
        pltpu.VMEM_SHARED((HR, W), jnp.float32),
        pltpu.SemaphoreType.DMA,
        pltpu.SemaphoreType.DMA,
        pltpu.SemaphoreType.DMA,
        pltpu.SemaphoreType.DMA,
    ],
)


def _final_body(h_ref, pp_ref, wr_ref, cb_ref,
                wir_ref, wiz_ref, win_ref, whr_ref, whz_ref, whn_ref,
                br_ref, bz_ref, bin_ref, bhn_ref, o_ref):
    h = h_ref[:, :H]
    p = pp_ref[...]
    cnt = jnp.maximum(p[:, H:H + 1], 1.0)
    agg = p[:, :H] / cnt
    conv = agg + _dot(h, wr_ref[...]) + cb_ref[...]
    m = jnp.maximum(conv, 0.0)
    r = jax.nn.sigmoid(_dot(m, wir_ref[...]) + _dot(h, whr_ref[...]) + br_ref[...])
    z = jax.nn.sigmoid(_dot(m, wiz_ref[...]) + _dot(h, whz_ref[...]) + bz_ref[...])
    n = jnp.tanh(_dot(m, win_ref[...]) + bin_ref[...]
                 + r * (_dot(h, whn_ref[...]) + bhn_ref[...]))
    o_ref[...] = (1.0 - z) * n + z * h


def _compute_final(h, aggp, weights):
    h_spec = pl.BlockSpec((TN, W), lambda i: (i, 0))
    pp_spec = pl.BlockSpec((TN, W), lambda i: (i, 0))
    w_spec = pl.BlockSpec((H, H), lambda i: (0, 0))
    b_spec = pl.BlockSpec((1, H), lambda i: (0, 0))
    return pl.pallas_call(
        _final_body,
        grid=(N // TN,),
        in_specs=[h_spec, pp_spec,
                  w_spec, b_spec,
                  w_spec, w_spec, w_spec, w_spec, w_spec, w_spec,
                  b_spec, b_spec, b_spec, b_spec],
        out_specs=pl.BlockSpec((TN, H), lambda i: (i, 0)),
        out_shape=jax.ShapeDtypeStruct((N, H), jnp.float32),
    )(h, aggp, *weights)


def kernel(x, edge_index, edge_attr, W_in, b_in, W1, b1, W2, b2,
           W_root, conv_bias, W_ih, W_hh, b_ih, b_hh):
    src2d = edge_index[0].reshape(NCHUNK, CH)
    dst2d = edge_index[1].reshape(NCHUNK, CH)

    W1r = jnp.repeat(W1, H, axis=1).astype(jnp.bfloat16)
    ea_bf = edge_attr.astype(jnp.bfloat16)
    b1r = jnp.repeat(b1, H).reshape(1, H2)
    P2 = W2.reshape(H2, H).astype(jnp.bfloat16)
    B2m = b2.reshape(H, H)

    Wir, Wiz, Win = (W_ih[i * H:(i + 1) * H, :].T for i in range(3))
    Whr, Whz, Whn = (W_hh[i * H:(i + 1) * H, :].T for i in range(3))
    br = (b_ih[0:H] + b_hh[0:H]).reshape(1, H)
    bz = (b_ih[H:2 * H] + b_hh[H:2 * H]).reshape(1, H)
    bin_ = b_ih[2 * H:3 * H].reshape(1, H)
    bhn = b_hh[2 * H:3 * H].reshape(1, H)

    h = _compute_h(x, W_in, b_in)
    hsrc = _gather_call(h, src2d)
    msg = _compute_msg(ea_bf, hsrc, W1r, b1r, P2, B2m)
    zrows = jnp.zeros((ZR2, W), jnp.float32)
    aggp = _scatter_call(dst2d, msg, zrows)

    weights = (W_root, conv_bias.reshape(1, H),
               Wir, Wiz, Win, Whr, Whz, Whn, br, bz, bin_, bhn)
    return _compute_final(h, aggp, weights)


# speedup vs baseline: 5.3679x; 1.0592x over previous
import jax
import jax.numpy as jnp
from jax import lax
from jax.experimental import pallas as pl
from jax.experimental.pallas import tpu as pltpu
from jax.experimental.pallas import tpu_sc as plsc

N = 10000
E = 160000
NODE_IN = 128
H = 32
H2 = H * H
W = 128

TN = 2000
TE = 1280

NC = 2
NS = 16
NW = NC * NS
CH = 128
NCHUNK = E // CH
CPW = -(-NCHUNK // NW)
NP = 10240

_PREC = lax.Precision.DEFAULT


def _dot(a, b):
    return jnp.dot(a, b, preferred_element_type=jnp.float32, precision=_PREC)


def _h_body(x_ref, w_ref, b_ref, o_ref):
    o_ref[...] = jnp.maximum(_dot(x_ref[...], w_ref[...]) + b_ref[...], 0.0)


def _compute_h(x, W_in, b_in):
    return pl.pallas_call(
        _h_body,
        grid=(N // TN,),
        in_specs=[
            pl.BlockSpec((TN, NODE_IN), lambda i: (i, 0)),
            pl.BlockSpec((NODE_IN, H), lambda i: (0, 0)),
            pl.BlockSpec((1, H), lambda i: (0, 0)),
        ],
        out_specs=pl.BlockSpec((TN, H), lambda i: (i, 0)),
        out_shape=jax.ShapeDtypeStruct((N, H), jnp.float32),
    )(x, W_in, b_in.reshape(1, H))


_SC_MESH = plsc.VectorSubcoreMesh(core_axis_name="c", subcore_axis_name="s")


def _gather_body(h_hbm, src_hbm, out_hbm,
                 idxA, idxB, rowsA, rowsB, sgA, sgB, swA, swB):
    wid = lax.axis_index("s") * NC + lax.axis_index("c")

    def chunk_of(j):
        return wid + j * NW

    def step(j, idx_v, rows_v, semg, semw):
        c = chunk_of(j)

        @pl.when((j >= 2) & (c < NCHUNK))
        def _():
            pltpu.make_async_copy(
                rows_v, out_hbm.at[pl.ds(0, CH), pl.ds(0, H)], semw).wait()

        @pl.when(c < NCHUNK)
        def _():
            pltpu.sync_copy(src_hbm.at[c], idx_v)
            pltpu.async_copy(h_hbm.at[idx_v], rows_v, semg).wait()
            pltpu.async_copy(
                rows_v, out_hbm.at[pl.ds(c * CH, CH), pl.ds(0, H)], semw)

    def body(k, carry):
        step(2 * k, idxA, rowsA, sgA, swA)
        step(2 * k + 1, idxB, rowsB, sgB, swB)
        return carry

    lax.fori_loop(0, CPW // 2, body, 0)

    for j, rows_v, semw in ((CPW - 2, rowsA, swA), (CPW - 1, rowsB, swB)):
        @pl.when(chunk_of(j) < NCHUNK)
        def _():
            pltpu.make_async_copy(
                rows_v, out_hbm.at[pl.ds(0, CH), pl.ds(0, H)], semw).wait()


_gather_call = pl.kernel(
    _gather_body,
    out_type=jax.ShapeDtypeStruct((E, W), jnp.float32),
    mesh=_SC_MESH,
    scratch_types=[
        pltpu.VMEM((CH,), jnp.int32),
        pltpu.VMEM((CH,), jnp.int32),
        pltpu.VMEM((CH, H), jnp.float32),
        pltpu.VMEM((CH, H), jnp.float32),
        pltpu.SemaphoreType.DMA,
        pltpu.SemaphoreType.DMA,
        pltpu.SemaphoreType.DMA,
        pltpu.SemaphoreType.DMA,
    ],
    compiler_params=pltpu.CompilerParams(use_tc_tiling_on_sc=False),
)


def _msg_body(ea_ref, hs_ref, w1r_ref, p2_ref, b2m_ref, o_ref):
    tbig = jnp.maximum(
        jnp.dot(ea_ref[...], w1r_ref[...],
                preferred_element_type=jnp.float32).astype(jnp.bfloat16), 0)
    hs = hs_ref[:, :H]
    u = tbig * jnp.tile(hs.astype(jnp.bfloat16), (1, H))
    msg = (jnp.dot(u, p2_ref[...], preferred_element_type=jnp.float32)
           + _dot(hs, b2m_ref[...]))
    o_ref[...] = jnp.concatenate(
        [msg,
         jnp.ones((TE, 1), jnp.float32),
         jnp.zeros((TE, W - H - 1), jnp.float32)], axis=1)


def _compute_msg(ea_aug, hsrc, W1r_aug, P2, B2m):
    return pl.pallas_call(
        _msg_body,
        grid=(E // TE,),
        in_specs=[
            pl.BlockSpec((TE, 17), lambda i: (i, 0)),
            pl.BlockSpec((TE, W), lambda i: (i, 0)),
            pl.BlockSpec((17, H2), lambda i: (0, 0)),
            pl.BlockSpec((H2, H), lambda i: (0, 0)),
            pl.BlockSpec((H, H), lambda i: (0, 0)),
        ],
        out_specs=pl.BlockSpec((TE, W), lambda i: (i, 0)),
        out_shape=jax.ShapeDtypeStruct((E, W), jnp.float32),
    )(ea_aug, hsrc, W1r_aug, P2, B2m)


HALF = NP // 2
HR = HALF + CH
ZR2 = HR // NS
OR2 = HALF // NS
CPW2 = -(-NCHUNK // NS)


def _clamp_idx(idx_ref, lo):
    for i in range(CH // 16):
        v = idx_ref[pl.ds(i * 16, 16)]
        rel = v - lo
        ok = (rel >= 0) & (rel < HALF)
        idx_ref[pl.ds(i * 16, 16)] = jnp.where(ok, rel, HALF)


def _scatter_body(dst_hbm, msg_hbm, zrows_hbm, agg_hbm,
                  idx0, idx1, idx2, rows0, rows1, rows2, acc_sh,
                  si0, si1, si2, sr0, sr1, sr2, ss0, ss1, ss2):
    cid = lax.axis_index("c")
    sid = lax.axis_index("s")
    lo = cid * HALF

    pltpu.sync_copy(zrows_hbm, acc_sh.at[pl.ds(sid * ZR2, ZR2)])
    plsc.subcore_barrier()

    def chunk_of(j):
        return sid + j * NS

    def start(j, idx_v, rows_v, semi, semr):
        c = chunk_of(j)

        @pl.when(c < NCHUNK)
        def _():
            pltpu.async_copy(dst_hbm.at[c], idx_v, semi)
            pltpu.async_copy(msg_hbm.at[pl.ds(c * CH, CH)], rows_v, semr)

    def process(j, idx_v, rows_v, semi, semr, sems):
        c = chunk_of(j)

        @pl.when(c < NCHUNK)
        def _():
            pltpu.make_async_copy(dst_hbm.at[c], idx_v, semi).wait()
            pltpu.make_async_copy(
                msg_hbm.at[pl.ds(c * CH, CH)], rows_v, semr).wait()
            _clamp_idx(idx_v, lo)
            pltpu.async_copy(rows_v, acc_sh.at[idx_v], sems, add=True)

    def drain(j, rows_v, sems):
        @pl.when((j >= 0) & (chunk_of(j) < NCHUNK))
        def _():
            pltpu.make_async_copy(
                rows_v, acc_sh.at[pl.ds(0, CH)], sems).wait()

    start(0, idx0, rows0, si0, sr0)
    start(1, idx1, rows1, si1, sr1)

    def body(k, carry):
        process(3 * k, idx0, rows0, si0, sr0, ss0)
        drain(3 * k - 1, rows2, ss2)
        start(3 * k + 2, idx2, rows2, si2, sr2)
        process(3 * k + 1, idx1, rows1, si1, sr1, ss1)
        drain(3 * k, rows0, ss0)
        start(3 * k + 3, idx0, rows0, si0, sr0)
        process(3 * k + 2, idx2, rows2, si2, sr2, ss2)
        drain(3 * k + 1, rows1, ss1)
        start(3 * k + 4, idx1, rows1, si1, sr1)
        return carry

    nit = -(-CPW2 // 3)
    lax.fori_loop(0, nit, body, 0)
    drain(3 * nit - 1, rows2, ss2)
    drain(3 * nit, rows0, ss0)
    drain(3 * nit + 1, rows1, ss1)
    plsc.subcore_barrier()

    pltpu.sync_copy(acc_sh.at[pl.ds(sid * OR2, OR2)],
                    agg_hbm.at[pl.ds(cid * HALF + sid * OR2, OR2)])


_scatter_call = pl.kernel(
    _scatter_body,
    out_type=jax.ShapeDtypeStruct((NP, W), jnp.float32),
    mesh=_SC_MESH,
    scratch_types=[
        pltpu.VMEM((CH,), jnp.int32),
        pltpu.VMEM((CH,), jnp.int32),
        pltpu.VMEM((CH,), jnp.int32),
        pltpu.VMEM((CH, W), jnp.float32),
        pltpu.VMEM((CH, W), jnp.float32),
        pltpu.VMEM((CH, W), jnp.float32),
        pltpu.VMEM_SHARED((HR, W), jnp.float32),
        pltpu.SemaphoreType.DMA,
        pltpu.SemaphoreType.DMA,
        pltpu.SemaphoreType.DMA,
        pltpu.SemaphoreType.DMA,
        pltpu.SemaphoreType.DMA,
        pltpu.SemaphoreType.DMA,
        pltpu.SemaphoreType.DMA,
        pltpu.SemaphoreType.DMA,
        pltpu.SemaphoreType.DMA,
    ],
)


def _final_body(h_ref, pp_ref, wr_ref, cb_ref,
                wir_ref, wiz_ref, win_ref, whr_ref, whz_ref, whn_ref,
                br_ref, bz_ref, bin_ref, bhn_ref, o_ref):
    h = h_ref[...]
    p = pp_ref[...]
    cnt = jnp.maximum(p[:, H:H + 1], 1.0)
    agg = p[:, :H] / cnt
    conv = agg + _dot(h, wr_ref[...]) + cb_ref[...]
    m = jnp.maximum(conv, 0.0)
    r = jax.nn.sigmoid(_dot(m, wir_ref[...]) + _dot(h, whr_ref[...]) + br_ref[...])
    z = jax.nn.sigmoid(_dot(m, wiz_ref[...]) + _dot(h, whz_ref[...]) + bz_ref[...])
    n = jnp.tanh(_dot(m, win_ref[...]) + bin_ref[...]
                 + r * (_dot(h, whn_ref[...]) + bhn_ref[...]))
    o_ref[...] = (1.0 - z) * n + z * h


def _compute_final(h, aggp, weights):
    h_spec = pl.BlockSpec((TN, H), lambda i: (i, 0))
    pp_spec = pl.BlockSpec((TN, W), lambda i: (i, 0))
    w_spec = pl.BlockSpec((H, H), lambda i: (0, 0))
    b_spec = pl.BlockSpec((1, H), lambda i: (0, 0))
    return pl.pallas_call(
        _final_body,
        grid=(N // TN,),
        in_specs=[h_spec, pp_spec,
                  w_spec, b_spec,
                  w_spec, w_spec, w_spec, w_spec, w_spec, w_spec,
                  b_spec, b_spec, b_spec, b_spec],
        out_specs=pl.BlockSpec((TN, H), lambda i: (i, 0)),
        out_shape=jax.ShapeDtypeStruct((N, H), jnp.float32),
    )(h, aggp, *weights)


def kernel(x, edge_index, edge_attr, W_in, b_in, W1, b1, W2, b2,
           W_root, conv_bias, W_ih, W_hh, b_ih, b_hh):
    src2d = edge_index[0].reshape(NCHUNK, CH)
    dst2d = edge_index[1].reshape(NCHUNK, CH)

    W1r_aug = jnp.concatenate(
        [jnp.repeat(W1, H, axis=1), jnp.repeat(b1, H).reshape(1, H2)],
        axis=0).astype(jnp.bfloat16)
    ea_aug = jnp.concatenate(
        [edge_attr, jnp.ones((E, 1), jnp.float32)], axis=1).astype(jnp.bfloat16)
    P2 = W2.reshape(H2, H).astype(jnp.bfloat16)
    B2m = b2.reshape(H, H)

    Wir, Wiz, Win = (W_ih[i * H:(i + 1) * H, :].T for i in range(3))
    Whr, Whz, Whn = (W_hh[i * H:(i + 1) * H, :].T for i in range(3))
    br = (b_ih[0:H] + b_hh[0:H]).reshape(1, H)
    bz = (b_ih[H:2 * H] + b_hh[H:2 * H]).reshape(1, H)
    bin_ = b_ih[2 * H:3 * H].reshape(1, H)
    bhn = b_hh[2 * H:3 * H].reshape(1, H)

    h = _compute_h(x, W_in, b_in)
    hsrc = _gather_call(h, src2d)
    msg = _compute_msg(ea_aug, hsrc, W1r_aug, P2, B2m)
    zrows = jnp.zeros((ZR2, W), jnp.float32)
    aggp = _scatter_call(dst2d, msg, zrows)

    weights = (W_root, conv_bias.reshape(1, H),
               Wir, Wiz, Win, Whr, Whz, Whn, br, bz, bin_, bhn)
    return _compute_final(h, aggp, weights)


# speedup vs baseline: 5.7264x; 1.0668x over previous
import jax
import jax.numpy as jnp
from jax import lax
from jax.experimental import pallas as pl
from jax.experimental.pallas import tpu as pltpu
from jax.experimental.pallas import tpu_sc as plsc

N = 10000
E = 160000
NODE_IN = 128
H = 32
H2 = H * H
W = 128

TN = 2000
TE = 1280

NC = 2
NS = 16
NW = NC * NS
CH = 128
NCHUNK = E // CH
CPW = -(-NCHUNK // NW)
NP = 10240

_PREC = lax.Precision.DEFAULT


def _dot(a, b):
    return jnp.dot(a, b, preferred_element_type=jnp.float32, precision=_PREC)


def _h_body(x_ref, w_ref, b_ref, o_ref):
    o_ref[...] = jnp.maximum(_dot(x_ref[...], w_ref[...]) + b_ref[...], 0.0)


def _compute_h(x, W_in, b_in):
    return pl.pallas_call(
        _h_body,
        grid=(N // TN,),
        in_specs=[
            pl.BlockSpec((TN, NODE_IN), lambda i: (i, 0)),
            pl.BlockSpec((NODE_IN, H), lambda i: (0, 0)),
            pl.BlockSpec((1, H), lambda i: (0, 0)),
        ],
        out_specs=pl.BlockSpec((TN, H), lambda i: (i, 0)),
        out_shape=jax.ShapeDtypeStruct((N, H), jnp.float32),
    )(x, W_in, b_in.reshape(1, H))


_SC_MESH = plsc.VectorSubcoreMesh(core_axis_name="c", subcore_axis_name="s")


def _gather_body(h_hbm, src_hbm, out_hbm,
                 idxA, idxB, rowsA, rowsB, sgA, sgB, swA, swB):
    wid = lax.axis_index("s") * NC + lax.axis_index("c")

    def chunk_of(j):
        return wid + j * NW

    def step(j, idx_v, rows_v, semg, semw):
        c = chunk_of(j)

        @pl.when((j >= 2) & (c < NCHUNK))
        def _():
            pltpu.make_async_copy(
                rows_v, out_hbm.at[pl.ds(0, CH), pl.ds(0, H)], semw).wait()

        @pl.when(c < NCHUNK)
        def _():
            pltpu.sync_copy(src_hbm.at[c], idx_v)
            pltpu.async_copy(h_hbm.at[idx_v], rows_v, semg).wait()
            pltpu.async_copy(
                rows_v, out_hbm.at[pl.ds(c * CH, CH), pl.ds(0, H)], semw)

    def body(k, carry):
        step(2 * k, idxA, rowsA, sgA, swA)
        step(2 * k + 1, idxB, rowsB, sgB, swB)
        return carry

    lax.fori_loop(0, CPW // 2, body, 0)

    for j, rows_v, semw in ((CPW - 2, rowsA, swA), (CPW - 1, rowsB, swB)):
        @pl.when(chunk_of(j) < NCHUNK)
        def _():
            pltpu.make_async_copy(
                rows_v, out_hbm.at[pl.ds(0, CH), pl.ds(0, H)], semw).wait()


_gather_call = pl.kernel(
    _gather_body,
    out_type=jax.ShapeDtypeStruct((E, W), jnp.float32),
    mesh=_SC_MESH,
    scratch_types=[
        pltpu.VMEM((CH,), jnp.int32),
        pltpu.VMEM((CH,), jnp.int32),
        pltpu.VMEM((CH, H), jnp.float32),
        pltpu.VMEM((CH, H), jnp.float32),
        pltpu.SemaphoreType.DMA,
        pltpu.SemaphoreType.DMA,
        pltpu.SemaphoreType.DMA,
        pltpu.SemaphoreType.DMA,
    ],
    compiler_params=pltpu.CompilerParams(use_tc_tiling_on_sc=False),
)


def _msg_body(ea_ref, hs_ref, w1r_ref, p2_ref, b2m_ref, o_ref):
    tbig = jnp.maximum(
        jnp.dot(ea_ref[...], w1r_ref[...],
                preferred_element_type=jnp.float32).astype(jnp.bfloat16), 0)
    hs = hs_ref[:, :H]
    u = tbig * jnp.tile(hs.astype(jnp.bfloat16), (1, H))
    msg = (jnp.dot(u, p2_ref[...], preferred_element_type=jnp.float32)
           + _dot(hs, b2m_ref[...]))
    o_ref[...] = jnp.concatenate(
        [msg,
         jnp.ones((TE, 1), jnp.float32),
         jnp.zeros((TE, W - H - 1), jnp.float32)], axis=1)


def _compute_msg(ea_aug, hsrc, W1r_aug, P2, B2m):
    return pl.pallas_call(
        _msg_body,
        grid=(E // TE,),
        in_specs=[
            pl.BlockSpec((TE, 17), lambda i: (i, 0)),
            pl.BlockSpec((TE, W), lambda i: (i, 0)),
            pl.BlockSpec((17, H2), lambda i: (0, 0)),
            pl.BlockSpec((H2, H), lambda i: (0, 0)),
            pl.BlockSpec((H, H), lambda i: (0, 0)),
        ],
        out_specs=pl.BlockSpec((TE, W), lambda i: (i, 0)),
        out_shape=jax.ShapeDtypeStruct((E, W), jnp.float32),
    )(ea_aug, hsrc, W1r_aug, P2, B2m)


SW = 48
HALF = NP // 2
HR = HALF + CH
ZR2 = HR // NS
OR2 = HALF // NS
CPW2 = -(-NCHUNK // NS)


def _clamp_idx(idx_ref, lo):
    for i in range(CH // 16):
        v = idx_ref[pl.ds(i * 16, 16)]
        rel = v - lo
        ok = (rel >= 0) & (rel < HALF)
        idx_ref[pl.ds(i * 16, 16)] = jnp.where(ok, rel, HALF)


def _scatter_body(dst_hbm, msg_hbm, zrows_hbm, agg_hbm,
                  idx0, idx1, idx2, rows0, rows1, rows2, acc_sh,
                  si0, si1, si2, sr0, sr1, sr2, ss0, ss1, ss2):
    cid = lax.axis_index("c")
    sid = lax.axis_index("s")
    lo = cid * HALF

    pltpu.sync_copy(zrows_hbm, acc_sh.at[pl.ds(sid * ZR2, ZR2)])
    plsc.subcore_barrier()

    def chunk_of(j):
        return sid + j * NS

    def start(j, idx_v, rows_v, semi, semr):
        c = chunk_of(j)

        @pl.when(c < NCHUNK)
        def _():
            pltpu.async_copy(dst_hbm.at[c], idx_v, semi)
            pltpu.async_copy(
                msg_hbm.at[pl.ds(c * CH, CH), pl.ds(0, SW)], rows_v, semr)

    def process(j, idx_v, rows_v, semi, semr, sems):
        c = chunk_of(j)

        @pl.when(c < NCHUNK)
        def _():
            pltpu.make_async_copy(dst_hbm.at[c], idx_v, semi).wait()
            pltpu.make_async_copy(
                msg_hbm.at[pl.ds(c * CH, CH), pl.ds(0, SW)], rows_v, semr).wait()
            _clamp_idx(idx_v, lo)
            pltpu.async_copy(rows_v, acc_sh.at[idx_v], sems, add=True)

    def drain(j, rows_v, sems):
        @pl.when((j >= 0) & (chunk_of(j) < NCHUNK))
        def _():
            pltpu.make_async_copy(
                rows_v, acc_sh.at[pl.ds(0, CH)], sems).wait()

    start(0, idx0, rows0, si0, sr0)
    start(1, idx1, rows1, si1, sr1)

    def body(k, carry):
        process(3 * k, idx0, rows0, si0, sr0, ss0)
        drain(3 * k - 1, rows2, ss2)
        start(3 * k + 2, idx2, rows2, si2, sr2)
        process(3 * k + 1, idx1, rows1, si1, sr1, ss1)
        drain(3 * k, rows0, ss0)
        start(3 * k + 3, idx0, rows0, si0, sr0)
        process(3 * k + 2, idx2, rows2, si2, sr2, ss2)
        drain(3 * k + 1, rows1, ss1)
        start(3 * k + 4, idx1, rows1, si1, sr1)
        return carry

    nit = -(-CPW2 // 3)
    lax.fori_loop(0, nit, body, 0)
    drain(3 * nit - 1, rows2, ss2)
    drain(3 * nit, rows0, ss0)
    drain(3 * nit + 1, rows1, ss1)
    plsc.subcore_barrier()

    pltpu.sync_copy(acc_sh.at[pl.ds(sid * OR2, OR2)],
                    agg_hbm.at[pl.ds(cid * HALF + sid * OR2, OR2), pl.ds(0, SW)])


_scatter_call = pl.kernel(
    _scatter_body,
    out_type=jax.ShapeDtypeStruct((NP, W), jnp.float32),
    mesh=_SC_MESH,
    scratch_types=[
        pltpu.VMEM((CH,), jnp.int32),
        pltpu.VMEM((CH,), jnp.int32),
        pltpu.VMEM((CH,), jnp.int32),
        pltpu.VMEM((CH, SW), jnp.float32),
        pltpu.VMEM((CH, SW), jnp.float32),
        pltpu.VMEM((CH, SW), jnp.float32),
        pltpu.VMEM_SHARED((HR, SW), jnp.float32),
        pltpu.SemaphoreType.DMA,
        pltpu.SemaphoreType.DMA,
        pltpu.SemaphoreType.DMA,
        pltpu.SemaphoreType.DMA,
        pltpu.SemaphoreType.DMA,
        pltpu.SemaphoreType.DMA,
        pltpu.SemaphoreType.DMA,
        pltpu.SemaphoreType.DMA,
        pltpu.SemaphoreType.DMA,
    ],
    compiler_params=pltpu.CompilerParams(use_tc_tiling_on_sc=False),
)


def _final_body(h_ref, pp_ref, wr_ref, cb_ref,
                wir_ref, wiz_ref, win_ref, whr_ref, whz_ref, whn_ref,
                br_ref, bz_ref, bin_ref, bhn_ref, o_ref):
    h = h_ref[...]
    p = pp_ref[...]
    cnt = jnp.maximum(p[:, H:H + 1], 1.0)
    agg = p[:, :H] / cnt
    conv = agg + _dot(h, wr_ref[...]) + cb_ref[...]
    m = jnp.maximum(conv, 0.0)
    r = jax.nn.sigmoid(_dot(m, wir_ref[...]) + _dot(h, whr_ref[...]) + br_ref[...])
    z = jax.nn.sigmoid(_dot(m, wiz_ref[...]) + _dot(h, whz_ref[...]) + bz_ref[...])
    n = jnp.tanh(_dot(m, win_ref[...]) + bin_ref[...]
                 + r * (_dot(h, whn_ref[...]) + bhn_ref[...]))
    o_ref[...] = (1.0 - z) * n + z * h


def _compute_final(h, aggp, weights):
    h_spec = pl.BlockSpec((TN, H), lambda i: (i, 0))
    pp_spec = pl.BlockSpec((TN, W), lambda i: (i, 0))
    w_spec = pl.BlockSpec((H, H), lambda i: (0, 0))
    b_spec = pl.BlockSpec((1, H), lambda i: (0, 0))
    return pl.pallas_call(
        _final_body,
        grid=(N // TN,),
        in_specs=[h_spec, pp_spec,
                  w_spec, b_spec,
                  w_spec, w_spec, w_spec, w_spec, w_spec, w_spec,
                  b_spec, b_spec, b_spec, b_spec],
        out_specs=pl.BlockSpec((TN, H), lambda i: (i, 0)),
        out_shape=jax.ShapeDtypeStruct((N, H), jnp.float32),
    )(h, aggp, *weights)


def kernel(x, edge_index, edge_attr, W_in, b_in, W1, b1, W2, b2,
           W_root, conv_bias, W_ih, W_hh, b_ih, b_hh):
    src2d = edge_index[0].reshape(NCHUNK, CH)
    dst2d = edge_index[1].reshape(NCHUNK, CH)

    W1r_aug = jnp.concatenate(
        [jnp.repeat(W1, H, axis=1), jnp.repeat(b1, H).reshape(1, H2)],
        axis=0).astype(jnp.bfloat16)
    ea_aug = jnp.concatenate(
        [edge_attr, jnp.ones((E, 1), jnp.float32)], axis=1).astype(jnp.bfloat16)
    P2 = W2.reshape(H2, H).astype(jnp.bfloat16)
    B2m = b2.reshape(H, H)

    Wir, Wiz, Win = (W_ih[i * H:(i + 1) * H, :].T for i in range(3))
    Whr, Whz, Whn = (W_hh[i * H:(i + 1) * H, :].T for i in range(3))
    br = (b_ih[0:H] + b_hh[0:H]).reshape(1, H)
    bz = (b_ih[H:2 * H] + b_hh[H:2 * H]).reshape(1, H)
    bin_ = b_ih[2 * H:3 * H].reshape(1, H)
    bhn = b_hh[2 * H:3 * H].reshape(1, H)

    h = _compute_h(x, W_in, b_in)
    hsrc = _gather_call(h, src2d)
    msg = _compute_msg(ea_aug, hsrc, W1r_aug, P2, B2m)
    zrows = jnp.zeros((ZR2, SW), jnp.float32)
    aggp = _scatter_call(dst2d, msg, zrows)

    weights = (W_root, conv_bias.reshape(1, H),
               Wir, Wiz, Win, Whr, Whz, Whn, br, bz, bin_, bhn)
    return _compute_final(h, aggp, weights)


# speedup vs baseline: 5.9265x; 1.0350x over previous
import jax
import jax.numpy as jnp
from jax import lax
from jax.experimental import pallas as pl
from jax.experimental.pallas import tpu as pltpu
from jax.experimental.pallas import tpu_sc as plsc

N = 10000
E = 160000
NODE_IN = 128
H = 32
H2 = H * H
W = 128

TN = 2000
TE = 2000

NC = 2
NS = 16
NW = NC * NS
CH = 128
NCHUNK = E // CH
CPW = -(-NCHUNK // NW)
NP = 10240

_PREC = lax.Precision.DEFAULT


def _dot(a, b):
    return jnp.dot(a, b, preferred_element_type=jnp.float32, precision=_PREC)


def _h_body(x_ref, w_ref, b_ref, o_ref):
    o_ref[...] = jnp.maximum(_dot(x_ref[...], w_ref[...]) + b_ref[...], 0.0)


def _compute_h(x, W_in, b_in):
    return pl.pallas_call(
        _h_body,
        grid=(N // TN,),
        in_specs=[
            pl.BlockSpec((TN, NODE_IN), lambda i: (i, 0)),
            pl.BlockSpec((NODE_IN, H), lambda i: (0, 0)),
            pl.BlockSpec((1, H), lambda i: (0, 0)),
        ],
        out_specs=pl.BlockSpec((TN, H), lambda i: (i, 0)),
        out_shape=jax.ShapeDtypeStruct((N, H), jnp.float32),
    )(x, W_in, b_in.reshape(1, H))


_SC_MESH = plsc.VectorSubcoreMesh(core_axis_name="c", subcore_axis_name="s")


def _gather_body(h_hbm, src_hbm, out_hbm,
                 idxA, idxB, rowsA, rowsB, sgA, sgB, swA, swB):
    wid = lax.axis_index("s") * NC + lax.axis_index("c")

    def chunk_of(j):
        return wid + j * NW

    def step(j, idx_v, rows_v, semg, semw):
        c = chunk_of(j)

        @pl.when((j >= 2) & (c < NCHUNK))
        def _():
            pltpu.make_async_copy(
                rows_v, out_hbm.at[pl.ds(0, CH), pl.ds(0, H)], semw).wait()

        @pl.when(c < NCHUNK)
        def _():
            pltpu.sync_copy(src_hbm.at[c], idx_v)
            pltpu.async_copy(h_hbm.at[idx_v], rows_v, semg).wait()
            pltpu.async_copy(
                rows_v, out_hbm.at[pl.ds(c * CH, CH), pl.ds(0, H)], semw)

    def body(k, carry):
        step(2 * k, idxA, rowsA, sgA, swA)
        step(2 * k + 1, idxB, rowsB, sgB, swB)
        return carry

    lax.fori_loop(0, CPW // 2, body, 0)

    for j, rows_v, semw in ((CPW - 2, rowsA, swA), (CPW - 1, rowsB, swB)):
        @pl.when(chunk_of(j) < NCHUNK)
        def _():
            pltpu.make_async_copy(
                rows_v, out_hbm.at[pl.ds(0, CH), pl.ds(0, H)], semw).wait()


_gather_call = pl.kernel(
    _gather_body,
    out_type=jax.ShapeDtypeStruct((E, W), jnp.float32),
    mesh=_SC_MESH,
    scratch_types=[
        pltpu.VMEM((CH,), jnp.int32),
        pltpu.VMEM((CH,), jnp.int32),
        pltpu.VMEM((CH, H), jnp.float32),
        pltpu.VMEM((CH, H), jnp.float32),
        pltpu.SemaphoreType.DMA,
        pltpu.SemaphoreType.DMA,
        pltpu.SemaphoreType.DMA,
        pltpu.SemaphoreType.DMA,
    ],
    compiler_params=pltpu.CompilerParams(use_tc_tiling_on_sc=False),
)


def _msg_body(ea_ref, hs_ref, w1r_ref, p2_ref, b2m_ref, o_ref):
    tbig = jnp.maximum(
        jnp.dot(ea_ref[...], w1r_ref[...],
                preferred_element_type=jnp.float32).astype(jnp.bfloat16), 0)
    hs = hs_ref[:, :H]
    u = tbig * jnp.tile(hs.astype(jnp.bfloat16), (1, H))
    msg = (jnp.dot(u, p2_ref[...], preferred_element_type=jnp.float32)
           + _dot(hs, b2m_ref[...]))
    o_ref[...] = jnp.concatenate(
        [msg,
         jnp.ones((TE, 1), jnp.float32),
         jnp.zeros((TE, W - H - 1), jnp.float32)], axis=1)


def _compute_msg(ea_aug, hsrc, W1r_aug, P2, B2m):
    return pl.pallas_call(
        _msg_body,
        grid=(E // TE,),
        in_specs=[
            pl.BlockSpec((TE, 17), lambda i: (i, 0)),
            pl.BlockSpec((TE, W), lambda i: (i, 0)),
            pl.BlockSpec((17, H2), lambda i: (0, 0)),
            pl.BlockSpec((H2, H), lambda i: (0, 0)),
            pl.BlockSpec((H, H), lambda i: (0, 0)),
        ],
        out_specs=pl.BlockSpec((TE, W), lambda i: (i, 0)),
        out_shape=jax.ShapeDtypeStruct((E, W), jnp.float32),
    )(ea_aug, hsrc, W1r_aug, P2, B2m)


SW = 48
HALF = NP // 2
HR = HALF + CH
ZR2 = HR // NS
OR2 = HALF // NS
CPW2 = -(-NCHUNK // NS)


def _clamp_idx(idx_ref, lo):
    for i in range(CH // 16):
        v = idx_ref[pl.ds(i * 16, 16)]
        rel = v - lo
        ok = (rel >= 0) & (rel < HALF)
        idx_ref[pl.ds(i * 16, 16)] = jnp.where(ok, rel, HALF)


def _scatter_body(dst_hbm, msg_hbm, zrows_hbm, agg_hbm,
                  idx0, idx1, idx2, rows0, rows1, rows2, acc_sh,
                  si0, si1, si2, sr0, sr1, sr2, ss0, ss1, ss2):
    cid = lax.axis_index("c")
    sid = lax.axis_index("s")
    lo = cid * HALF

    pltpu.sync_copy(zrows_hbm, acc_sh.at[pl.ds(sid * ZR2, ZR2)])
    plsc.subcore_barrier()

    def chunk_of(j):
        return sid + j * NS

    def start(j, idx_v, rows_v, semi, semr):
        c = chunk_of(j)

        @pl.when(c < NCHUNK)
        def _():
            pltpu.async_copy(dst_hbm.at[c], idx_v, semi)
            pltpu.async_copy(
                msg_hbm.at[pl.ds(c * CH, CH), pl.ds(0, SW)], rows_v, semr)

    def process(j, idx_v, rows_v, semi, semr, sems):
        c = chunk_of(j)

        @pl.when(c < NCHUNK)
        def _():
            pltpu.make_async_copy(dst_hbm.at[c], idx_v, semi).wait()
            pltpu.make_async_copy(
                msg_hbm.at[pl.ds(c * CH, CH), pl.ds(0, SW)], rows_v, semr).wait()
            _clamp_idx(idx_v, lo)
            pltpu.async_copy(rows_v, acc_sh.at[idx_v], sems, add=True)

    def drain(j, rows_v, sems):
        @pl.when((j >= 0) & (chunk_of(j) < NCHUNK))
        def _():
            pltpu.make_async_copy(
                rows_v, acc_sh.at[pl.ds(0, CH)], sems).wait()

    start(0, idx0, rows0, si0, sr0)
    start(1, idx1, rows1, si1, sr1)

    def body(k, carry):
        process(3 * k, idx0, rows0, si0, sr0, ss0)
        drain(3 * k - 1, rows2, ss2)
        start(3 * k + 2, idx2, rows2, si2, sr2)
        process(3 * k + 1, idx1, rows1, si1, sr1, ss1)
        drain(3 * k, rows0, ss0)
        start(3 * k + 3, idx0, rows0, si0, sr0)
        process(3 * k + 2, idx2, rows2, si2, sr2, ss2)
        drain(3 * k + 1, rows1, ss1)
        start(3 * k + 4, idx1, rows1, si1, sr1)
        return carry

    nit = -(-CPW2 // 3)
    lax.fori_loop(0, nit, body, 0)
    drain(3 * nit - 1, rows2, ss2)
    drain(3 * nit, rows0, ss0)
    drain(3 * nit + 1, rows1, ss1)
    plsc.subcore_barrier()

    pltpu.sync_copy(acc_sh.at[pl.ds(sid * OR2, OR2)],
                    agg_hbm.at[pl.ds(cid * HALF + sid * OR2, OR2), pl.ds(0, SW)])


_scatter_call = pl.kernel(
    _scatter_body,
    out_type=jax.ShapeDtypeStruct((NP, W), jnp.float32),
    mesh=_SC_MESH,
    scratch_types=[
        pltpu.VMEM((CH,), jnp.int32),
        pltpu.VMEM((CH,), jnp.int32),
        pltpu.VMEM((CH,), jnp.int32),
        pltpu.VMEM((CH, SW), jnp.float32),
        pltpu.VMEM((CH, SW), jnp.float32),
        pltpu.VMEM((CH, SW), jnp.float32),
        pltpu.VMEM_SHARED((HR, SW), jnp.float32),
        pltpu.SemaphoreType.DMA,
        pltpu.SemaphoreType.DMA,
        pltpu.SemaphoreType.DMA,
        pltpu.SemaphoreType.DMA,
        pltpu.SemaphoreType.DMA,
        pltpu.SemaphoreType.DMA,
        pltpu.SemaphoreType.DMA,
        pltpu.SemaphoreType.DMA,
        pltpu.SemaphoreType.DMA,
    ],
    compiler_params=pltpu.CompilerParams(use_tc_tiling_on_sc=False),
)


def _final_body(h_ref, pp_ref, wr_ref, cb_ref,
                wir_ref, wiz_ref, win_ref, whr_ref, whz_ref, whn_ref,
                br_ref, bz_ref, bin_ref, bhn_ref, o_ref):
    h = h_ref[...]
    p = pp_ref[...]
    cnt = jnp.maximum(p[:, H:H + 1], 1.0)
    agg = p[:, :H] / cnt
    conv = agg + _dot(h, wr_ref[...]) + cb_ref[...]
    m = jnp.maximum(conv, 0.0)
    r = jax.nn.sigmoid(_dot(m, wir_ref[...]) + _dot(h, whr_ref[...]) + br_ref[...])
    z = jax.nn.sigmoid(_dot(m, wiz_ref[...]) + _dot(h, whz_ref[...]) + bz_ref[...])
    n = jnp.tanh(_dot(m, win_ref[...]) + bin_ref[...]
                 + r * (_dot(h, whn_ref[...]) + bhn_ref[...]))
    o_ref[...] = (1.0 - z) * n + z * h


def _compute_final(h, aggp, weights):
    h_spec = pl.BlockSpec((TN, H), lambda i: (i, 0))
    pp_spec = pl.BlockSpec((TN, W), lambda i: (i, 0))
    w_spec = pl.BlockSpec((H, H), lambda i: (0, 0))
    b_spec = pl.BlockSpec((1, H), lambda i: (0, 0))
    return pl.pallas_call(
        _final_body,
        grid=(N // TN,),
        in_specs=[h_spec, pp_spec,
                  w_spec, b_spec,
                  w_spec, w_spec, w_spec, w_spec, w_spec, w_spec,
                  b_spec, b_spec, b_spec, b_spec],
        out_specs=pl.BlockSpec((TN, H), lambda i: (i, 0)),
        out_shape=jax.ShapeDtypeStruct((N, H), jnp.float32),
    )(h, aggp, *weights)


def kernel(x, edge_index, edge_attr, W_in, b_in, W1, b1, W2, b2,
           W_root, conv_bias, W_ih, W_hh, b_ih, b_hh):
    src2d = edge_index[0].reshape(NCHUNK, CH)
    dst2d = edge_index[1].reshape(NCHUNK, CH)

    W1r_aug = jnp.concatenate(
        [jnp.repeat(W1, H, axis=1), jnp.repeat(b1, H).reshape(1, H2)],
        axis=0).astype(jnp.bfloat16)
    ea_aug = jnp.concatenate(
        [edge_attr, jnp.ones((E, 1), jnp.float32)], axis=1).astype(jnp.bfloat16)
    P2 = W2.reshape(H2, H).astype(jnp.bfloat16)
    B2m = b2.reshape(H, H)

    Wir, Wiz, Win = (W_ih[i * H:(i + 1) * H, :].T for i in range(3))
    Whr, Whz, Whn = (W_hh[i * H:(i + 1) * H, :].T for i in range(3))
    br = (b_ih[0:H] + b_hh[0:H]).reshape(1, H)
    bz = (b_ih[H:2 * H] + b_hh[H:2 * H]).reshape(1, H)
    bin_ = b_ih[2 * H:3 * H].reshape(1, H)
    bhn = b_hh[2 * H:3 * H].reshape(1, H)

    h = _compute_h(x, W_in, b_in)
    hsrc = _gather_call(h, src2d)
    msg = _compute_msg(ea_aug, hsrc, W1r_aug, P2, B2m)
    zrows = jnp.zeros((ZR2, SW), jnp.float32)
    aggp = _scatter_call(dst2d, msg, zrows)

    weights = (W_root, conv_bias.reshape(1, H),
               Wir, Wiz, Win, Whr, Whz, Whn, br, bz, bin_, bhn)
    return _compute_final(h, aggp, weights)
